# baseline (speedup 1.0000x reference)
import functools

import jax
import jax.numpy as jnp
from jax import lax
from jax.experimental import pallas as pl
from jax.experimental.pallas import tpu as pltpu
from jax.experimental.pallas import tpu_sc as plsc

N = 10000
E = 320000
D_IN = 128
H = 22
OUT = 11

NC = 2
NS = 16
NW = NC * NS

NP = 10240
CPR = NP // NS
SUB = 80
ROWS2D = E // SUB
SPW = ROWS2D // NW

_f32 = jnp.float32
_i32 = jnp.int32


def _bcast_lane(vec, k):
  idx = jnp.full((16,), k, dtype=_i32)
  return jnp.take_along_axis(vec, idx, axis=0)


def _make_edge_scatter(wp, do_gather):
  mesh = plsc.VectorSubcoreMesh(
      core_axis_name="c", subcore_axis_name="s", num_cores=NC, num_subcores=NS
  )
  scratch = []
  if do_gather:
    scratch.append(pltpu.VMEM_SHARED((NP, wp), _f32))
  scratch += [
      pltpu.VMEM_SHARED((NP, wp), _f32),
      pltpu.VMEM((SPW, SUB), _i32),
      pltpu.VMEM((SPW, SUB), _f32),
      pltpu.VMEM((2, SUB, wp), _f32),
  ]
  if do_gather:
    scratch += [
        pltpu.VMEM((SPW, SUB), _i32),
        pltpu.VMEM((2, SUB, wp), _f32),
    ]
  scratch += [pltpu.SemaphoreType.DMA] * 4

  def body(*refs):
    if do_gather:
      (table, row2d, col2d, ew2d, zeros_hbm, out,
       tab_sp, acc_sp, cidx_v, ew_v, val_v, ridx_v, rows_v,
       gsem0, gsem1, ssem0, ssem1) = refs
    else:
      (col2d, ew2d, zeros_hbm, out,
       acc_sp, cidx_v, ew_v, val_v,
       gsem0, gsem1, ssem0, ssem1) = refs
    c = lax.axis_index("c")
    s = lax.axis_index("s")
    w = c * NS + s
    rs = s * CPR
    if do_gather:
      pltpu.sync_copy(table.at[pl.ds(rs, CPR)], tab_sp.at[pl.ds(rs, CPR)])
    pltpu.sync_copy(zeros_hbm.at[pl.ds(rs, CPR)], acc_sp.at[pl.ds(rs, CPR)])
    pltpu.sync_copy(col2d.at[w], cidx_v)
    pltpu.sync_copy(ew2d.at[w], ew_v)
    if do_gather:
      pltpu.sync_copy(row2d.at[w], ridx_v)
    plsc.subcore_barrier()

    gsems = (gsem0, gsem1)
    ssems = (ssem0, ssem1)

    def gather_start(j, buf):
      if do_gather:
        pltpu.async_copy(tab_sp.at[ridx_v.at[j]], rows_v.at[buf], gsems[buf])

    def gather_copy(j, buf):
      return pltpu.make_async_copy(
          tab_sp.at[ridx_v.at[j]], rows_v.at[buf], gsems[buf]
      )

    def scale(j, buf):
      ew_row = ew_v.at[j]
      vbuf = val_v.at[buf]
      if do_gather:
        rbuf = rows_v.at[buf]
      for g in range(SUB // 16):
        ew_vec = ew_row[pl.ds(g * 16, 16)]
        for k in range(16):
          sv = _bcast_lane(ew_vec, k)
          r = g * 16 + k
          val_r = vbuf.at[r]
          if do_gather:
            rows_r = rbuf.at[r]
            for h in range(wp // 16):
              val_r[pl.ds(h * 16, 16)] = rows_r[pl.ds(h * 16, 16)] * sv
          else:
            val_r[pl.ds(0, 16)] = sv

    def scatter_start(j, buf):
      pltpu.async_copy(val_v.at[buf], acc_sp.at[cidx_v.at[j]], ssems[buf],
                       add=True)

    def scatter_wait(j, buf):
      pltpu.make_async_copy(
          val_v.at[buf], acc_sp.at[cidx_v.at[j]], ssems[buf]
      ).wait()

    def process(j, buf, first):
      if do_gather:
        gather_copy(j, buf).wait()
      if not first:
        scatter_wait(j, buf)
      scale(j, buf)
      scatter_start(j, buf)

    gather_start(0, 0)

    def step(i, carry):
      j0 = 2 * i
      gather_start(j0 + 1, 1)

      @pl.when(i == 0)
      def _():
        process(j0, 0, True)

      @pl.when(i > 0)
      def _():
        process(j0, 0, False)

      gather_start(j0 + 2, 0)

      @pl.when(i == 0)
      def _():
        process(j0 + 1, 1, True)

      @pl.when(i > 0)
      def _():
        process(j0 + 1, 1, False)
      return carry

    lax.fori_loop(0, (SPW - 1) // 2, step, 0)
    process(SPW - 1, 0, False)
    scatter_wait(SPW - 1, 0)
    scatter_wait(SPW - 2, 1)
    plsc.subcore_barrier()
    pltpu.sync_copy(acc_sp.at[pl.ds(rs, CPR)], out.at[c].at[pl.ds(rs, CPR)])

  return pl.kernel(
      body,
      out_type=jax.ShapeDtypeStruct((NC, NP, wp), _f32),
      mesh=mesh,
      compiler_params=pltpu.CompilerParams(
          use_tc_tiling_on_sc=False,
          disable_bounds_checks=True,
          disable_semaphore_checks=True,
          skip_device_barrier=True,
      ),
      scratch_types=scratch,
  )


_sc_edge32 = _make_edge_scatter(32, True)
_sc_edge16 = _make_edge_scatter(16, True)
_sc_degree = _make_edge_scatter(16, False)


def _relu(v):
  return jnp.maximum(v, 0.0)


def _tc0_body(d0, d1, x_r, w_r, u_r, wl_r, bl_r,
              xw_o, xwp_o, dis_o, dis2_o, ub_o):
  deg = d0[:, 0:1] + d1[:, 0:1] + 1.0
  dis = jnp.where(deg > 0, lax.rsqrt(jnp.maximum(deg, 1e-12)), 0.0)
  dis2 = dis * dis
  xw = jnp.dot(x_r[...], w_r[...], preferred_element_type=_f32)
  xw_o[...] = xw
  xwp_o[...] = xw * dis
  dis_o[...] = dis
  dis2_o[...] = dis2
  ub_o[...] = _relu(
      jnp.dot(u_r[...], wl_r[...], preferred_element_type=_f32) + bl_r[...]
  )


def _tc0(d0, d1, xp, w1p, up, wlp, blp):
  return pl.pallas_call(
      _tc0_body,
      out_shape=(
          jax.ShapeDtypeStruct((NP, 32), _f32),
          jax.ShapeDtypeStruct((NP, 32), _f32),
          jax.ShapeDtypeStruct((NP, 1), _f32),
          jax.ShapeDtypeStruct((NP, 1), _f32),
          jax.ShapeDtypeStruct((1, 32), _f32),
      ),
  )(d0, d1, xp, w1p, up, wlp, blp)


def _tc_mid_body(a0, a1, xw_r, dis_r, dis2_r, b_r, wn_r, ex_r, xw_o, xwp_o):
  dis = dis_r[...]
  h = _relu(dis * (a0[...] + a1[...]) + dis2_r[...] * xw_r[...] + b_r[...])
  h = h + ex_r[...]
  xwn = jnp.dot(h, wn_r[...], preferred_element_type=_f32)
  xw_o[...] = xwn
  xwp_o[...] = xwn * dis


def _tc_mid(wo, a0, a1, xw, dis, dis2, bp, wnp, extra):
  return pl.pallas_call(
      _tc_mid_body,
      out_shape=(
          jax.ShapeDtypeStruct((NP, wo), _f32),
          jax.ShapeDtypeStruct((NP, wo), _f32),
      ),
  )(a0, a1, xw, dis, dis2, bp, wnp, extra)


def _tc_fin_body(a0, a1, xw_r, dis_r, dis2_r, b_r, m_r, o_ref):
  o_ref[...] = (
      dis_r[...] * (a0[...] + a1[...])
      + dis2_r[...] * xw_r[...]
      + b_r[...]
      + (m_r[...] - 1.0) * 1000.0
  )


def _tc_fin(a0, a1, xw, dis, dis2, b4p, maskp):
  return pl.pallas_call(
      _tc_fin_body,
      out_shape=jax.ShapeDtypeStruct((NP, 16), _f32),
  )(a0, a1, xw, dis, dis2, b4p, maskp)


def kernel(x, edge_index, edge_attr, u, action_mask,
           W1, b1, W2, b2, W3, b3, W4, b4, Wl, bl):
  row2d = edge_index[0].reshape(NW, SPW, SUB)
  col2d = edge_index[1].reshape(NW, SPW, SUB)
  ew2d = edge_attr.reshape(NW, SPW, SUB)

  xp = jnp.pad(x, ((0, NP - N), (0, 0)))
  w1p = jnp.pad(W1, ((0, 0), (0, 32 - H)))
  w2p = jnp.pad(W2, ((0, 32 - H), (0, 32 - H)))
  w3p = jnp.pad(W3, ((0, 32 - H), (0, 32 - H)))
  w4p = jnp.pad(W4, ((0, 32 - H), (0, 16 - OUT)))
  wlp = jnp.pad(Wl, ((0, 32 - H), (0, 32 - H)))
  up = jnp.pad(u, ((0, 0), (0, 32 - H)))
  b1p = jnp.pad(b1, (0, 32 - H)).reshape(1, 32)
  b2p = jnp.pad(b2, (0, 32 - H)).reshape(1, 32)
  b3p = jnp.pad(b3, (0, 32 - H)).reshape(1, 32)
  b4p = jnp.pad(b4, (0, 16 - OUT)).reshape(1, 16)
  blp = jnp.pad(bl, (0, 32 - H)).reshape(1, 32)
  maskp = jnp.pad(action_mask, ((0, NP - N), (0, 16 - OUT)),
                  constant_values=1.0)
  zeros32 = jnp.zeros((NP, 32), _f32)
  zeros16 = jnp.zeros((NP, 16), _f32)
  zrow32 = jnp.zeros((1, 32), _f32)

  degp = _sc_degree(col2d, ew2d, zeros16)
  xw1, xwp1, dis, dis2, ub = _tc0(degp[0], degp[1], xp, w1p, up, wlp, blp)

  acc1 = _sc_edge32(xwp1, row2d, col2d, ew2d, zeros32)
  xw2, xwp2 = _tc_mid(32, acc1[0], acc1[1], xw1, dis, dis2, b1p, w2p, ub)

  acc2 = _sc_edge32(xwp2, row2d, col2d, ew2d, zeros32)
  xw3, xwp3 = _tc_mid(32, acc2[0], acc2[1], xw2, dis, dis2, b2p, w3p, zrow32)

  acc3 = _sc_edge32(xwp3, row2d, col2d, ew2d, zeros32)
  xw4, xwp4 = _tc_mid(16, acc3[0], acc3[1], xw3, dis, dis2, b3p, w4p, zrow32)

  acc4 = _sc_edge16(xwp4, row2d, col2d, ew2d, zeros16)
  outp = _tc_fin(acc4[0], acc4[1], xw4, dis, dis2, b4p, maskp)

  return outp[:N, :OUT]


# speedup vs baseline: 1.4450x; 1.4450x over previous
import functools

import jax
import jax.numpy as jnp
from jax import lax
from jax.experimental import pallas as pl
from jax.experimental.pallas import tpu as pltpu
from jax.experimental.pallas import tpu_sc as plsc

N = 10000
E = 320000
D_IN = 128
H = 22
OUT = 11

NC = 2
NS = 16
NW = NC * NS

NP = 10240
CPR = NP // NS
SUB = 80
ROWS2D = E // SUB
SPW = ROWS2D // NW

_f32 = jnp.float32
_i32 = jnp.int32


def _bcast_lane(vec, k):
  idx = jnp.full((16,), k, dtype=_i32)
  return jnp.take_along_axis(vec, idx, axis=0)


def _make_edge_scatter(wp, do_gather):
  mesh = plsc.VectorSubcoreMesh(
      core_axis_name="c", subcore_axis_name="s", num_cores=NC, num_subcores=NS
  )
  scratch = []
  if do_gather:
    scratch.append(pltpu.VMEM_SHARED((NP, wp), _f32))
  scratch += [
      pltpu.VMEM_SHARED((NP, wp), _f32),
      pltpu.VMEM((SPW, SUB), _i32),
      pltpu.VMEM((SPW, SUB), _f32),
      pltpu.VMEM((2, SUB, wp), _f32),
  ]
  if do_gather:
    scratch += [
        pltpu.VMEM((SPW, SUB), _i32),
        pltpu.VMEM((2, SUB, wp), _f32),
    ]
  scratch += [pltpu.SemaphoreType.DMA] * 4

  def body(*refs):
    if do_gather:
      (table, row2d, col2d, ew2d, zeros_hbm, out,
       tab_sp, acc_sp, cidx_v, ew_v, val_v, ridx_v, rows_v,
       gsem0, gsem1, ssem0, ssem1) = refs
    else:
      (col2d, ew2d, zeros_hbm, out,
       acc_sp, cidx_v, ew_v, val_v,
       gsem0, gsem1, ssem0, ssem1) = refs
    c = lax.axis_index("c")
    s = lax.axis_index("s")
    w = c * NS + s
    rs = s * CPR
    if do_gather:
      pltpu.sync_copy(table.at[pl.ds(rs, CPR)], tab_sp.at[pl.ds(rs, CPR)])
    pltpu.sync_copy(zeros_hbm.at[pl.ds(rs, CPR)], acc_sp.at[pl.ds(rs, CPR)])
    pltpu.sync_copy(col2d.at[w], cidx_v)
    pltpu.sync_copy(ew2d.at[w], ew_v)
    if do_gather:
      pltpu.sync_copy(row2d.at[w], ridx_v)
    plsc.subcore_barrier()

    gsems = (gsem0, gsem1)
    ssems = (ssem0, ssem1)

    def gather_start(j, buf):
      if do_gather:
        pltpu.async_copy(tab_sp.at[ridx_v.at[j]], rows_v.at[buf], gsems[buf])

    def gather_copy(j, buf):
      return pltpu.make_async_copy(
          tab_sp.at[ridx_v.at[j]], rows_v.at[buf], gsems[buf]
      )

    def scale(j, buf):
      ew_row = ew_v.at[j]
      vbuf = val_v.at[buf]
      if do_gather:
        rbuf = rows_v.at[buf]
      for g in range(SUB // 16):
        ew_vec = ew_row[pl.ds(g * 16, 16)]
        for k in range(16):
          sv = _bcast_lane(ew_vec, k)
          r = g * 16 + k
          val_r = vbuf.at[r]
          if do_gather:
            rows_r = rbuf.at[r]
            for h in range(wp // 16):
              val_r[pl.ds(h * 16, 16)] = rows_r[pl.ds(h * 16, 16)] * sv
          else:
            for h in range(wp // 16):
              val_r[pl.ds(h * 16, 16)] = sv

    def scatter_start(j, buf):
      pltpu.async_copy(val_v.at[buf], acc_sp.at[cidx_v.at[j]], ssems[buf],
                       add=True)

    def scatter_wait(j, buf):
      pltpu.make_async_copy(
          val_v.at[buf], acc_sp.at[cidx_v.at[j]], ssems[buf]
      ).wait()

    def process(j, buf, first):
      if do_gather:
        gather_copy(j, buf).wait()
      if not first:
        scatter_wait(j, buf)
      scale(j, buf)
      scatter_start(j, buf)

    gather_start(0, 0)

    def step(i, carry):
      j0 = 2 * i
      gather_start(j0 + 1, 1)

      @pl.when(i == 0)
      def _():
        process(j0, 0, True)

      @pl.when(i > 0)
      def _():
        process(j0, 0, False)

      gather_start(j0 + 2, 0)

      @pl.when(i == 0)
      def _():
        process(j0 + 1, 1, True)

      @pl.when(i > 0)
      def _():
        process(j0 + 1, 1, False)
      return carry

    lax.fori_loop(0, (SPW - 1) // 2, step, 0)
    process(SPW - 1, 0, False)
    scatter_wait(SPW - 1, 0)
    scatter_wait(SPW - 2, 1)
    plsc.subcore_barrier()
    pltpu.sync_copy(acc_sp.at[pl.ds(rs, CPR)], out.at[c].at[pl.ds(rs, CPR)])

  return pl.kernel(
      body,
      out_type=jax.ShapeDtypeStruct((NC, NP, wp), _f32),
      mesh=mesh,
      compiler_params=pltpu.CompilerParams(
          use_tc_tiling_on_sc=False,
          disable_bounds_checks=True,
          disable_semaphore_checks=True,
      ),
      scratch_types=scratch,
  )


_sc_edge32 = _make_edge_scatter(32, True)
_sc_degree = _make_edge_scatter(32, False)

NPK = NP // 4


def _relu(v):
  return jnp.maximum(v, 0.0)


def _tc0_body(dp_r, xg_r, w_r, u_r, wl_r, bl_r,
              xw_o, xwp_o, dis_o, dis2_o, ub_o):
  deg = dp_r[0:NPK, :] + dp_r[NPK:2 * NPK, :] + 1.0
  dis = jnp.where(deg > 0, lax.rsqrt(jnp.maximum(deg, 1e-12)), 0.0)
  dis2 = dis * dis
  xw = jnp.dot(xg_r[...], w_r[...], preferred_element_type=_f32)
  xw_o[...] = xw
  xwp_o[...] = xw * dis
  dis_o[...] = dis
  dis2_o[...] = dis2
  ub = _relu(
      jnp.dot(u_r[...], wl_r[...], preferred_element_type=_f32) + bl_r[...]
  )
  ub_o[...] = jnp.concatenate([ub, ub, ub, ub], axis=1)


def _tc0(dp, xg, w1bd, up, wlp, blp):
  return pl.pallas_call(
      _tc0_body,
      out_shape=(
          jax.ShapeDtypeStruct((NPK, 128), _f32),
          jax.ShapeDtypeStruct((NPK, 128), _f32),
          jax.ShapeDtypeStruct((NPK, 128), _f32),
          jax.ShapeDtypeStruct((NPK, 128), _f32),
          jax.ShapeDtypeStruct((1, 128), _f32),
      ),
  )(dp, xg, w1bd, up, wlp, blp)


def _tc_mid_body(ap_r, xw_r, dis_r, dis2_r, b_r, wn_r, ex_r, xw_o, xwp_o):
  dis = dis_r[...]
  h = _relu(dis * (ap_r[0:NPK, :] + ap_r[NPK:2 * NPK, :])
            + dis2_r[...] * xw_r[...] + b_r[...])
  h = h + ex_r[...]
  xwn = jnp.dot(h, wn_r[...], preferred_element_type=_f32)
  xw_o[...] = xwn
  xwp_o[...] = xwn * dis


def _tc_mid(ap, xw, dis, dis2, bp, wnbd, extra):
  return pl.pallas_call(
      _tc_mid_body,
      out_shape=(
          jax.ShapeDtypeStruct((NPK, 128), _f32),
          jax.ShapeDtypeStruct((NPK, 128), _f32),
      ),
  )(ap, xw, dis, dis2, bp, wnbd, extra)


def _tc_fin_body(ap_r, xw_r, dis_r, dis2_r, b_r, m_r, o_ref):
  o_ref[...] = (
      dis_r[...] * (ap_r[0:NPK, :] + ap_r[NPK:2 * NPK, :])
      + dis2_r[...] * xw_r[...]
      + b_r[...]
      + (m_r[...] - 1.0) * 1000.0
  )


def _tc_fin(ap, xw, dis, dis2, b4p, maskp):
  return pl.pallas_call(
      _tc_fin_body,
      out_shape=jax.ShapeDtypeStruct((NPK, 128), _f32),
  )(ap, xw, dis, dis2, b4p, maskp)


def kernel(x, edge_index, edge_attr, u, action_mask,
           W1, b1, W2, b2, W3, b3, W4, b4, Wl, bl):
  row2d = edge_index[0].reshape(NW, SPW, SUB)
  col2d = edge_index[1].reshape(NW, SPW, SUB)
  ew2d = edge_attr.reshape(NW, SPW, SUB)

  eye4 = jnp.eye(4, dtype=_f32)
  xg = jnp.pad(x, ((0, NP - N), (0, 0))).reshape(NPK, 4 * D_IN)
  w1bd = jnp.kron(eye4, jnp.pad(W1, ((0, 0), (0, 32 - H))))
  w2bd = jnp.kron(eye4, jnp.pad(W2, ((0, 32 - H), (0, 32 - H))))
  w3bd = jnp.kron(eye4, jnp.pad(W3, ((0, 32 - H), (0, 32 - H))))
  w4bd = jnp.kron(eye4, jnp.pad(W4, ((0, 32 - H), (0, 32 - OUT))))
  wlp = jnp.pad(Wl, ((0, 32 - H), (0, 32 - H)))
  up = jnp.pad(u, ((0, 0), (0, 32 - H)))
  b1t = jnp.tile(jnp.pad(b1, (0, 32 - H)), 4).reshape(1, 128)
  b2t = jnp.tile(jnp.pad(b2, (0, 32 - H)), 4).reshape(1, 128)
  b3t = jnp.tile(jnp.pad(b3, (0, 32 - H)), 4).reshape(1, 128)
  b4t = jnp.tile(jnp.pad(b4, (0, 32 - OUT)), 4).reshape(1, 128)
  blp = jnp.pad(bl, (0, 32 - H)).reshape(1, 32)
  maskp = jnp.pad(action_mask, ((0, NP - N), (0, 32 - OUT)),
                  constant_values=1.0).reshape(NPK, 128)
  zeros32 = jnp.zeros((NP, 32), _f32)
  zrow = jnp.zeros((1, 128), _f32)

  degp = _sc_degree(col2d, ew2d, zeros32)
  xw1, xwp1, dis, dis2, ub = _tc0(degp.reshape(2 * NPK, 128), xg, w1bd,
                                  up, wlp, blp)

  acc1 = _sc_edge32(xwp1.reshape(NP, 32), row2d, col2d, ew2d, zeros32)
  xw2, xwp2 = _tc_mid(acc1.reshape(2 * NPK, 128), xw1, dis, dis2, b1t,
                      w2bd, ub)

  acc2 = _sc_edge32(xwp2.reshape(NP, 32), row2d, col2d, ew2d, zeros32)
  xw3, xwp3 = _tc_mid(acc2.reshape(2 * NPK, 128), xw2, dis, dis2, b2t,
                      w3bd, zrow)

  acc3 = _sc_edge32(xwp3.reshape(NP, 32), row2d, col2d, ew2d, zeros32)
  xw4, xwp4 = _tc_mid(acc3.reshape(2 * NPK, 128), xw3, dis, dis2, b3t,
                      w4bd, zrow)

  acc4 = _sc_edge32(xwp4.reshape(NP, 32), row2d, col2d, ew2d, zeros32)
  outp = _tc_fin(acc4.reshape(2 * NPK, 128), xw4, dis, dis2, b4t, maskp)

  return outp.reshape(NP, 32)[:N, :OUT]


# speedup vs baseline: 1.5172x; 1.0499x over previous
import functools

import jax
import jax.numpy as jnp
from jax import lax
from jax.experimental import pallas as pl
from jax.experimental.pallas import tpu as pltpu
from jax.experimental.pallas import tpu_sc as plsc

N = 10000
E = 320000
D_IN = 128
H = 22
OUT = 11

NC = 2
NS = 16
NW = NC * NS

NP = 10240
CPR = NP // NS
SUB = 80
ROWS2D = E // SUB
SPW = ROWS2D // NW

_f32 = jnp.float32
_i32 = jnp.int32


def _bcast_lane(vec, k):
  idx = jnp.full((16,), k, dtype=_i32)
  return jnp.take_along_axis(vec, idx, axis=0)


def _make_edge_scatter(wp, do_gather):
  mesh = plsc.VectorSubcoreMesh(
      core_axis_name="c", subcore_axis_name="s", num_cores=NC, num_subcores=NS
  )
  scratch = []
  if do_gather:
    scratch.append(pltpu.VMEM_SHARED((NP, wp), _f32))
  scratch += [
      pltpu.VMEM_SHARED((NP, wp), _f32),
      pltpu.VMEM((SPW, SUB), _i32),
      pltpu.VMEM((SPW, SUB), _f32),
      pltpu.VMEM((2, SUB, wp), _f32),
  ]
  if do_gather:
    scratch += [
        pltpu.VMEM((SPW, SUB), _i32),
        pltpu.VMEM((2, SUB, wp), _f32),
    ]
  scratch += [pltpu.SemaphoreType.DMA] * 4

  def body(*refs):
    if do_gather:
      (table, row2d, col2d, ew2d, zeros_hbm, out,
       tab_sp, acc_sp, cidx_v, ew_v, val_v, ridx_v, rows_v,
       gsem0, gsem1, ssem0, ssem1) = refs
    else:
      (col2d, ew2d, zeros_hbm, out,
       acc_sp, cidx_v, ew_v, val_v,
       gsem0, gsem1, ssem0, ssem1) = refs
    c = lax.axis_index("c")
    s = lax.axis_index("s")
    w = c * NS + s
    rs = s * CPR
    stage = []
    if do_gather:
      stage.append(pltpu.async_copy(
          table.at[pl.ds(rs, CPR)], tab_sp.at[pl.ds(rs, CPR)], gsem0))
      stage.append(pltpu.async_copy(row2d.at[w], ridx_v, gsem0))
    stage.append(pltpu.async_copy(
        zeros_hbm.at[pl.ds(rs, CPR)], acc_sp.at[pl.ds(rs, CPR)], gsem0))
    stage.append(pltpu.async_copy(col2d.at[w], cidx_v, gsem0))
    stage.append(pltpu.async_copy(ew2d.at[w], ew_v, gsem0))
    for cp in stage:
      cp.wait()
    plsc.subcore_barrier()

    gsems = (gsem0, gsem1)
    ssems = (ssem0, ssem1)

    def gather_start(j, buf):
      if do_gather:
        pltpu.async_copy(tab_sp.at[ridx_v.at[j]], rows_v.at[buf], gsems[buf])

    def gather_copy(j, buf):
      return pltpu.make_async_copy(
          tab_sp.at[ridx_v.at[j]], rows_v.at[buf], gsems[buf]
      )

    def scale(j, buf):
      ew_row = ew_v.at[j]
      vbuf = val_v.at[buf]
      if do_gather:
        rbuf = rows_v.at[buf]
      for g in range(SUB // 16):
        ew_vec = ew_row[pl.ds(g * 16, 16)]
        for k in range(16):
          sv = _bcast_lane(ew_vec, k)
          r = g * 16 + k
          val_r = vbuf.at[r]
          if do_gather:
            rows_r = rbuf.at[r]
            for h in range(wp // 16):
              val_r[pl.ds(h * 16, 16)] = rows_r[pl.ds(h * 16, 16)] * sv
          else:
            for h in range(wp // 16):
              val_r[pl.ds(h * 16, 16)] = sv

    def scatter_start(j, buf):
      pltpu.async_copy(val_v.at[buf], acc_sp.at[cidx_v.at[j]], ssems[buf],
                       add=True)

    def scatter_wait(j, buf):
      pltpu.make_async_copy(
          val_v.at[buf], acc_sp.at[cidx_v.at[j]], ssems[buf]
      ).wait()

    def process(j, buf, first):
      if do_gather:
        gather_copy(j, buf).wait()
      if not first:
        scatter_wait(j, buf)
      scale(j, buf)
      scatter_start(j, buf)

    gather_start(0, 0)

    def step(i, carry):
      j0 = 2 * i
      gather_start(j0 + 1, 1)

      @pl.when(i == 0)
      def _():
        process(j0, 0, True)

      @pl.when(i > 0)
      def _():
        process(j0, 0, False)

      gather_start(j0 + 2, 0)

      @pl.when(i == 0)
      def _():
        process(j0 + 1, 1, True)

      @pl.when(i > 0)
      def _():
        process(j0 + 1, 1, False)
      return carry

    lax.fori_loop(0, (SPW - 1) // 2, step, 0)
    process(SPW - 1, 0, False)
    scatter_wait(SPW - 1, 0)
    scatter_wait(SPW - 2, 1)
    plsc.subcore_barrier()
    pltpu.sync_copy(acc_sp.at[pl.ds(rs, CPR)], out.at[c].at[pl.ds(rs, CPR)])

  return pl.kernel(
      body,
      out_type=jax.ShapeDtypeStruct((NC, NP, wp), _f32),
      mesh=mesh,
      compiler_params=pltpu.CompilerParams(
          use_tc_tiling_on_sc=False,
          disable_bounds_checks=True,
          disable_semaphore_checks=True,
      ),
      scratch_types=scratch,
  )


_sc_edge32 = _make_edge_scatter(32, True)
_sc_degree = _make_edge_scatter(32, False)

NPK = NP // 4


def _relu(v):
  return jnp.maximum(v, 0.0)


def _tc0_body(dp_r, xg_r, w_r, u_r, wl_r, bl_r,
              xw_o, xwp_o, dis_o, dis2_o, ub_o):
  deg = dp_r[0:NPK, :] + dp_r[NPK:2 * NPK, :] + 1.0
  dis = jnp.where(deg > 0, lax.rsqrt(jnp.maximum(deg, 1e-12)), 0.0)
  dis2 = dis * dis
  xw = jnp.dot(xg_r[...], w_r[...], preferred_element_type=_f32)
  xw_o[...] = xw
  xwp_o[...] = xw * dis
  dis_o[...] = dis
  dis2_o[...] = dis2
  ub = _relu(
      jnp.dot(u_r[...], wl_r[...], preferred_element_type=_f32) + bl_r[...]
  )
  ub_o[...] = jnp.concatenate([ub, ub, ub, ub], axis=1)


def _tc0(dp, xg, w1bd, up, wlp, blp):
  return pl.pallas_call(
      _tc0_body,
      out_shape=(
          jax.ShapeDtypeStruct((NPK, 128), _f32),
          jax.ShapeDtypeStruct((NPK, 128), _f32),
          jax.ShapeDtypeStruct((NPK, 128), _f32),
          jax.ShapeDtypeStruct((NPK, 128), _f32),
          jax.ShapeDtypeStruct((1, 128), _f32),
      ),
  )(dp, xg, w1bd, up, wlp, blp)


def _tc_mid_body(ap_r, xw_r, dis_r, dis2_r, b_r, wn_r, ex_r, xw_o, xwp_o):
  dis = dis_r[...]
  h = _relu(dis * (ap_r[0:NPK, :] + ap_r[NPK:2 * NPK, :])
            + dis2_r[...] * xw_r[...] + b_r[...])
  h = h + ex_r[...]
  xwn = jnp.dot(h, wn_r[...], preferred_element_type=_f32)
  xw_o[...] = xwn
  xwp_o[...] = xwn * dis


def _tc_mid(ap, xw, dis, dis2, bp, wnbd, extra):
  return pl.pallas_call(
      _tc_mid_body,
      out_shape=(
          jax.ShapeDtypeStruct((NPK, 128), _f32),
          jax.ShapeDtypeStruct((NPK, 128), _f32),
      ),
  )(ap, xw, dis, dis2, bp, wnbd, extra)


def _tc_fin_body(ap_r, xw_r, dis_r, dis2_r, b_r, m_r, o_ref):
  o_ref[...] = (
      dis_r[...] * (ap_r[0:NPK, :] + ap_r[NPK:2 * NPK, :])
      + dis2_r[...] * xw_r[...]
      + b_r[...]
      + (m_r[...] - 1.0) * 1000.0
  )


def _tc_fin(ap, xw, dis, dis2, b4p, maskp):
  return pl.pallas_call(
      _tc_fin_body,
      out_shape=jax.ShapeDtypeStruct((NPK, 128), _f32),
  )(ap, xw, dis, dis2, b4p, maskp)


def kernel(x, edge_index, edge_attr, u, action_mask,
           W1, b1, W2, b2, W3, b3, W4, b4, Wl, bl):
  row2d = edge_index[0].reshape(NW, SPW, SUB)
  col2d = edge_index[1].reshape(NW, SPW, SUB)
  ew2d = edge_attr.reshape(NW, SPW, SUB)

  eye4 = jnp.eye(4, dtype=_f32)
  xg = jnp.pad(x, ((0, NP - N), (0, 0))).reshape(NPK, 4 * D_IN)
  w1bd = jnp.kron(eye4, jnp.pad(W1, ((0, 0), (0, 32 - H))))
  w2bd = jnp.kron(eye4, jnp.pad(W2, ((0, 32 - H), (0, 32 - H))))
  w3bd = jnp.kron(eye4, jnp.pad(W3, ((0, 32 - H), (0, 32 - H))))
  w4bd = jnp.kron(eye4, jnp.pad(W4, ((0, 32 - H), (0, 32 - OUT))))
  wlp = jnp.pad(Wl, ((0, 32 - H), (0, 32 - H)))
  up = jnp.pad(u, ((0, 0), (0, 32 - H)))
  b1t = jnp.tile(jnp.pad(b1, (0, 32 - H)), 4).reshape(1, 128)
  b2t = jnp.tile(jnp.pad(b2, (0, 32 - H)), 4).reshape(1, 128)
  b3t = jnp.tile(jnp.pad(b3, (0, 32 - H)), 4).reshape(1, 128)
  b4t = jnp.tile(jnp.pad(b4, (0, 32 - OUT)), 4).reshape(1, 128)
  blp = jnp.pad(bl, (0, 32 - H)).reshape(1, 32)
  maskp = jnp.pad(action_mask, ((0, NP - N), (0, 32 - OUT)),
                  constant_values=1.0).reshape(NPK, 128)
  zeros32 = jnp.zeros((NP, 32), _f32)
  zrow = jnp.zeros((1, 128), _f32)

  degp = _sc_degree(col2d, ew2d, zeros32)
  xw1, xwp1, dis, dis2, ub = _tc0(degp.reshape(2 * NPK, 128), xg, w1bd,
                                  up, wlp, blp)

  acc1 = _sc_edge32(xwp1.reshape(NP, 32), row2d, col2d, ew2d, zeros32)
  xw2, xwp2 = _tc_mid(acc1.reshape(2 * NPK, 128), xw1, dis, dis2, b1t,
                      w2bd, ub)

  acc2 = _sc_edge32(xwp2.reshape(NP, 32), row2d, col2d, ew2d, zeros32)
  xw3, xwp3 = _tc_mid(acc2.reshape(2 * NPK, 128), xw2, dis, dis2, b2t,
                      w3bd, zrow)

  acc3 = _sc_edge32(xwp3.reshape(NP, 32), row2d, col2d, ew2d, zeros32)
  xw4, xwp4 = _tc_mid(acc3.reshape(2 * NPK, 128), xw3, dis, dis2, b3t,
                      w4bd, zrow)

  acc4 = _sc_edge32(xwp4.reshape(NP, 32), row2d, col2d, ew2d, zeros32)
  outp = _tc_fin(acc4.reshape(2 * NPK, 128), xw4, dis, dis2, b4t, maskp)

  return outp.reshape(NP, 32)[:N, :OUT]
